# baseline (speedup 1.0000x reference)
import functools

import jax
import jax.numpy as jnp
from jax import lax
from jax.experimental import pallas as pl
from jax.experimental.pallas import tpu as pltpu
from jax.experimental.pallas import tpu_sc as plsc

F32 = jnp.float32
I32 = jnp.int32

_NC = 2
_NS = 16
_NW = _NC * _NS


def _sc_mesh():
    return plsc.VectorSubcoreMesh(core_axis_name="c", subcore_axis_name="s",
                                  num_cores=_NC, num_subcores=_NS)


def _chunk_of(per_worker, cap=448):
    for c in range(min(per_worker, cap), 0, -1):
        if per_worker % c == 0 and c % 8 == 0:
            return c
    return per_worker


def _pad_rows(n):
    per = -(-n // (_NS * 8)) * 8
    return per * _NS


def _pick_block(n, target):
    b = min(n, target)
    while n % b or b % 8:
        b -= 1
    return b


def _ln(v):
    m = jnp.mean(v, axis=-1, keepdims=True)
    var = jnp.mean((v - m) ** 2, axis=-1, keepdims=True)
    return (v - m) * lax.rsqrt(var + 1e-5)


def _dot(a, b):
    return jnp.dot(a, b, preferred_element_type=F32)


def _k_node_enc(x_ref, wn_ref, bn_ref, deg_ref, h_ref, ent_ref):
    h = _ln(_dot(x_ref[...], wn_ref[...]) + bn_ref[...])
    h_ref[...] = h
    deg = deg_ref[:, 0]
    p = 1.0 / (deg + 1e-6)
    s = jnp.sum(deg * (-p * jnp.log(p + 1e-6)))
    ent_ref[0, 0, :] = jnp.broadcast_to(s, (ent_ref.shape[-1],))


def _k_edge_enc(ea_ref, we_ref, be_ref, e_ref):
    e_ref[...] = _ln(_dot(ea_ref[...], we_ref[...]) + be_ref[...])


def _k_node1(h_ref, agg_ref, deg_ref, ws_ref, wnn_ref, sb_ref,
             mac_ref, st_ref):
    h = h_ref[...]
    agg = (agg_ref[0] + agg_ref[1]) / (deg_ref[...] + 1e-6)
    mac = _dot(h, ws_ref[...]) + _dot(agg, wnn_ref[...]) + sb_ref[...]
    mac_ref[...] = mac
    st_ref[0, 0, :] = jnp.sum(mac, axis=0)
    st_ref[0, 1, :] = jnp.sum(mac * mac, axis=0)


def _k_edge_score(e_ref, hs_ref, hd_ref, we_ref, wq_ref, wk_ref, wv_ref,
                  g_ref, gt_ref, t_ref, sc_ref, mx_ref):
    ee = _dot(e_ref[...], we_ref[...])
    hs = hs_ref[...]
    q = _dot(hd_ref[...], wq_ref[...])
    k = _dot(hs, wk_ref[...]) + ee
    v = _dot(hs, wv_ref[...]) + ee
    sc = _dot(q * k, g_ref[...]) * 0.25
    t_ref[...] = v
    sc_ref[...] = sc
    mx8 = jnp.max(sc, axis=0, keepdims=True)
    mx_ref[0] = _dot(mx8, gt_ref[...])


def _k_edge_exp(t_ref, sc_ref, gmx_ref, gt_ref, msg_ref, z_ref):
    z8 = jnp.exp(sc_ref[...] - gmx_ref[...])
    z = _dot(z8, gt_ref[...])
    msg_ref[...] = z * t_ref[...]
    z_ref[...] = z


def _k_node2(s_ref, den_ref, h_ref, wo_ref, bo_ref, z_ref, st_ref):
    s = s_ref[0] + s_ref[1]
    den = den_ref[0] + den_ref[1] + 1e-16
    outn = _dot(s / den, wo_ref[...]) + bo_ref[...]
    z = outn + h_ref[...]
    z_ref[...] = z
    st_ref[0, 0, :] = jnp.sum(z, axis=0)
    st_ref[0, 1, :] = jnp.sum(z * z, axis=0)


def _k_node3(mac_ref, z_ref, mu_ref, var_ref, h_ref, ent_ref,
             a1_ref, eb_ref, w2_ref, b2_ref,
             hnew_ref, hmic_ref):
    mu = mu_ref[...]
    var = var_ref[...]
    hmac = jax.nn.gelu((mac_ref[...] - mu[0:1, :]) *
                       lax.rsqrt(var[0:1, :] + 1e-5)) + h_ref[...]
    hmic = jax.nn.gelu((z_ref[...] - mu[1:2, :]) *
                       lax.rsqrt(var[1:2, :] + 1e-5))
    dh2 = a1_ref.shape[1] // 2
    gi = (_dot(hmac, a1_ref[:, :dh2]) + _dot(hmic, a1_ref[:, dh2:])
          + ent_ref[0, 0] * eb_ref[0:1, :dh2] + eb_ref[1:2, :dh2])
    sg = jax.nn.silu(gi)
    a = jax.nn.sigmoid(_dot(sg, w2_ref[...]) + b2_ref[...])
    hnew_ref[...] = a * hmac + (1.0 - a) * hmic
    hmic_ref[...] = hmic


def _k_edge_mlp(hs_ref, hd_ref, e_ref, w1_ref, b1_ref, w2_ref, b2_ref,
                wg_ref, bg_ref, eo_ref):
    hs = hs_ref[...]
    hd = hd_ref[...]
    e = e_ref[...]
    d = e.shape[1]
    w1 = w1_ref[...]
    wg = wg_ref[...]
    c1 = (_dot(hs, w1[:d]) + _dot(hd, w1[d:2 * d]) + _dot(e, w1[2 * d:])
          + b1_ref[...])
    u = jax.nn.gelu(_ln(c1))
    upd = _dot(u, w2_ref[...]) + b2_ref[...]
    g = jax.nn.sigmoid(_dot(hs, wg[:d]) + _dot(hd, wg[d:2 * d])
                       + _dot(e, wg[2 * d:]) + bg_ref[...])
    eo_ref[...] = _ln(upd * g + e)


def _rows(block, ncols):
    return pl.BlockSpec((block, ncols), lambda i: (i, 0))


def _bcast(shape):
    nd = len(shape)
    return pl.BlockSpec(shape, lambda i: (0,) * nd)


def _call(body, grid, in_specs, out_specs, out_shapes, *args):
    r = pl.pallas_call(
        body,
        grid=(grid,),
        in_specs=in_specs,
        out_specs=out_specs,
        out_shape=out_shapes,
    )(*args)
    return r[0] if len(out_shapes) == 1 else r


def _sc_gather_scatter(h, src, dst, zeros):
    n, d = h.shape
    e = src.shape[0]
    ew = e // _NW
    c = _chunk_of(ew, cap=200)
    iters = ew // c
    n_pad = zeros.shape[0]
    rps = n_pad // _NS

    def body(h_hbm, src_hbm, dst_hbm, zer_hbm, hs_out, hd_out, agg_out,
             sidx, didx, rows, acc, sem):
        ci = lax.axis_index("c")
        si = lax.axis_index("s")
        wid = si * _NC + ci
        pltpu.sync_copy(zer_hbm.at[pl.ds(si * rps, rps)],
                        acc.at[pl.ds(si * rps, rps)])
        plsc.subcore_barrier()
        base0 = wid * ew

        def step(i, carry):
            b = base0 + i * c
            pltpu.sync_copy(src_hbm.at[pl.ds(b, c)], sidx)
            pltpu.sync_copy(dst_hbm.at[pl.ds(b, c)], didx)
            pltpu.async_copy(h_hbm.at[sidx], rows, sem).wait()
            pltpu.sync_copy(rows, hs_out.at[pl.ds(b, c)])
            pltpu.sync_copy(rows, acc.at[didx], add=True)
            pltpu.async_copy(h_hbm.at[didx], rows, sem).wait()
            pltpu.sync_copy(rows, hd_out.at[pl.ds(b, c)])
            return carry

        lax.fori_loop(0, iters, step, 0)
        plsc.subcore_barrier()
        pltpu.sync_copy(acc.at[pl.ds(si * rps, rps)],
                        agg_out.at[ci].at[pl.ds(si * rps, rps)])

    f = pl.kernel(
        body,
        out_type=[jax.ShapeDtypeStruct((e, d), F32),
                  jax.ShapeDtypeStruct((e, d), F32),
                  jax.ShapeDtypeStruct((_NC, n_pad, d), F32)],
        mesh=_sc_mesh(),
        scratch_types=[pltpu.VMEM((c,), I32), pltpu.VMEM((c,), I32),
                       pltpu.VMEM((c, d), F32),
                       pltpu.VMEM_SHARED((n_pad, d), F32),
                       pltpu.SemaphoreType.DMA],
    )
    return f(h, src, dst, zeros)


def _sc_scatter(p, dst, zeros):
    n_pad, w = zeros.shape
    e, _ = p.shape
    ew = e // _NW
    c = _chunk_of(ew, cap=200)
    iters = ew // c
    rps = n_pad // _NS

    def body(p_hbm, dst_hbm, zer_hbm, out, pbuf0, didx0, didx1,
             acc, s0, si0, si1):
        ci = lax.axis_index("c")
        si = lax.axis_index("s")
        wid = si * _NC + ci
        pltpu.sync_copy(zer_hbm.at[pl.ds(si * rps, rps)],
                        acc.at[pl.ds(si * rps, rps)])
        plsc.subcore_barrier()
        base0 = wid * ew

        def step(j, carry):
            b0 = base0 + (2 * j) * c
            b1 = b0 + c
            ai0 = pltpu.async_copy(dst_hbm.at[pl.ds(b0, c)], didx0, si0)
            ai1 = pltpu.async_copy(dst_hbm.at[pl.ds(b1, c)], didx1, si1)
            ap0 = pltpu.async_copy(p_hbm.at[pl.ds(b0, c)], pbuf0, s0)
            ai0.wait()
            ap0.wait()
            pltpu.sync_copy(pbuf0, acc.at[didx0], add=True)
            ap1 = pltpu.async_copy(p_hbm.at[pl.ds(b1, c)], pbuf0, s0)
            ai1.wait()
            ap1.wait()
            pltpu.sync_copy(pbuf0, acc.at[didx1], add=True)
            return carry

        lax.fori_loop(0, iters // 2, step, 0)
        plsc.subcore_barrier()
        pltpu.sync_copy(acc.at[pl.ds(si * rps, rps)],
                        out.at[ci].at[pl.ds(si * rps, rps)])

    f = pl.kernel(
        body,
        out_type=[jax.ShapeDtypeStruct((_NC, n_pad, w), F32)],
        mesh=_sc_mesh(),
        scratch_types=[pltpu.VMEM((c, w), F32),
                       pltpu.VMEM((c,), I32), pltpu.VMEM((c,), I32),
                       pltpu.VMEM_SHARED((n_pad, w), F32),
                       pltpu.SemaphoreType.DMA,
                       pltpu.SemaphoreType.DMA, pltpu.SemaphoreType.DMA],
    )
    return f(p, dst, zeros)[0]


def _sc_dual_gather(tab, src, dst):
    n, d = tab.shape
    e = src.shape[0]
    ew = e // _NW
    c = _chunk_of(ew, cap=200)
    iters = ew // c

    def body(t_hbm, src_hbm, dst_hbm, s_out, d_out,
             sidx0, didx0, sidx1, didx1, r00, r01, r10, r11,
             g00, g01, g10, g11, i00, i01, i10, i11):
        ci = lax.axis_index("c")
        si = lax.axis_index("s")
        wid = si * _NC + ci
        base0 = wid * ew

        def step(j, carry):
            b0 = base0 + (2 * j) * c
            b1 = b0 + c
            a00 = pltpu.async_copy(src_hbm.at[pl.ds(b0, c)], sidx0, i00)
            a01 = pltpu.async_copy(dst_hbm.at[pl.ds(b0, c)], didx0, i01)
            a10 = pltpu.async_copy(src_hbm.at[pl.ds(b1, c)], sidx1, i10)
            a11 = pltpu.async_copy(dst_hbm.at[pl.ds(b1, c)], didx1, i11)
            a00.wait()
            c00 = pltpu.async_copy(t_hbm.at[sidx0], r00, g00)
            a01.wait()
            c01 = pltpu.async_copy(t_hbm.at[didx0], r01, g01)
            a10.wait()
            c10 = pltpu.async_copy(t_hbm.at[sidx1], r10, g10)
            a11.wait()
            c11 = pltpu.async_copy(t_hbm.at[didx1], r11, g11)
            c00.wait()
            pltpu.sync_copy(r00, s_out.at[pl.ds(b0, c)])
            c01.wait()
            pltpu.sync_copy(r01, d_out.at[pl.ds(b0, c)])
            c10.wait()
            pltpu.sync_copy(r10, s_out.at[pl.ds(b1, c)])
            c11.wait()
            pltpu.sync_copy(r11, d_out.at[pl.ds(b1, c)])
            return carry

        lax.fori_loop(0, iters // 2, step, 0)

    f = pl.kernel(
        body,
        out_type=[jax.ShapeDtypeStruct((e, d), F32),
                  jax.ShapeDtypeStruct((e, d), F32)],
        mesh=_sc_mesh(),
        scratch_types=[pltpu.VMEM((c,), I32), pltpu.VMEM((c,), I32),
                       pltpu.VMEM((c,), I32), pltpu.VMEM((c,), I32),
                       pltpu.VMEM((c, d), F32), pltpu.VMEM((c, d), F32),
                       pltpu.VMEM((c, d), F32), pltpu.VMEM((c, d), F32),
                       pltpu.SemaphoreType.DMA, pltpu.SemaphoreType.DMA,
                       pltpu.SemaphoreType.DMA, pltpu.SemaphoreType.DMA,
                       pltpu.SemaphoreType.DMA, pltpu.SemaphoreType.DMA,
                       pltpu.SemaphoreType.DMA, pltpu.SemaphoreType.DMA],
    )
    return f(tab, src, dst)


def kernel(x, edge_index, edge_attr, Wn, bn, We_enc, be_enc, sage_Ws,
           sage_Wn, sage_b, WQ, WK, WV, WE, Wout, bout, Wm1, bm1, Wm2,
           bm2, Wg, bg, Wsg1, bsg1, Wsg2, bsg2):
    n, d = x.shape
    e_cnt = edge_index.shape[1]
    nheads = 8
    dh = d // nheads
    src = edge_index[0]
    dst = edge_index[1]

    bn_rows = _pick_block(n, 1024)
    be_rows = _pick_block(e_cnt, 2048)
    gn = n // bn_rows
    ge = e_cnt // be_rows

    hid = jnp.arange(d, dtype=jnp.int32) // dh
    g_sum = (hid[:, None] == jnp.arange(nheads)[None, :]).astype(F32)
    g_bc = g_sum.T

    n_pad = _pad_rows(n)
    z128 = jnp.zeros((n_pad, d), F32)
    ones_e = jnp.ones((e_cnt,), F32)
    deg_src = jax.ops.segment_sum(ones_e, src, n).reshape(n, 1)
    deg_dst = jax.ops.segment_sum(ones_e, dst, n).reshape(n, 1)

    h, ent_part = _call(
        _k_node_enc, gn,
        [_rows(bn_rows, d), _bcast((d, d)), _bcast((1, d)),
         _rows(bn_rows, 1)],
        [_rows(bn_rows, d), pl.BlockSpec((1, 1, d), lambda i: (i, 0, 0))],
        [jax.ShapeDtypeStruct((n, d), F32),
         jax.ShapeDtypeStruct((gn, 1, d), F32)],
        x, Wn, bn.reshape(1, d), deg_src)
    graph_entropy = jnp.sum(ent_part[:, 0, 0]) / e_cnt

    ed = edge_attr.shape[1]
    e = _call(
        _k_edge_enc, ge,
        [_rows(be_rows, ed), _bcast((ed, d)), _bcast((1, d))],
        [_rows(be_rows, d)],
        [jax.ShapeDtypeStruct((e_cnt, d), F32)],
        edge_attr, We_enc, be_enc.reshape(1, d))

    nlayers = sage_Ws.shape[0]
    for l in range(nlayers):
        h_src, h_dst = _sc_dual_gather(h, src, dst)
        agg_p = _sc_scatter(h_src, dst, z128)
        mac, st = _call(
            _k_node1, gn,
            [_rows(bn_rows, d),
             pl.BlockSpec((2, bn_rows, d), lambda i: (0, i, 0)),
             _rows(bn_rows, 1),
             _bcast((d, d)), _bcast((d, d)), _bcast((1, d))],
            [_rows(bn_rows, d),
             pl.BlockSpec((1, 2, d), lambda i: (i, 0, 0))],
            [jax.ShapeDtypeStruct((n, d), F32),
             jax.ShapeDtypeStruct((gn, 2, d), F32)],
            h, agg_p, deg_dst, sage_Ws[l], sage_Wn[l],
            sage_b[l].reshape(1, d))
        mu1 = jnp.sum(st[:, 0, :], axis=0) / n
        var1 = jnp.sum(st[:, 1, :], axis=0) / n - mu1 * mu1

        t, sc, mxp = _call(
            _k_edge_score, ge,
            [_rows(be_rows, d), _rows(be_rows, d), _rows(be_rows, d),
             _bcast((d, d)), _bcast((d, d)), _bcast((d, d)),
             _bcast((d, d)),
             _bcast((d, nheads)), _bcast((nheads, d))],
            [_rows(be_rows, d), _rows(be_rows, nheads),
             pl.BlockSpec((1, 1, d), lambda i: (i, 0, 0))],
            [jax.ShapeDtypeStruct((e_cnt, d), F32),
             jax.ShapeDtypeStruct((e_cnt, nheads), F32),
             jax.ShapeDtypeStruct((ge, 1, d), F32)],
            e, h_src, h_dst, WE[l], WQ[l], WK[l], WV[l], g_sum, g_bc)
        gmx8 = jnp.max(mxp[:, 0, ::dh], axis=0).reshape(1, nheads)
        msg, zx = _call(
            _k_edge_exp, ge,
            [_rows(be_rows, d), _rows(be_rows, nheads),
             _bcast((1, nheads)), _bcast((nheads, d))],
            [_rows(be_rows, d), _rows(be_rows, d)],
            [jax.ShapeDtypeStruct((e_cnt, d), F32),
             jax.ShapeDtypeStruct((e_cnt, d), F32)],
            t, sc, gmx8, g_bc)

        s_acc = _sc_scatter(msg, dst, z128)
        den_p = _sc_scatter(zx, dst, z128)

        z, st2 = _call(
            _k_node2, gn,
            [pl.BlockSpec((2, bn_rows, d), lambda i: (0, i, 0)),
             pl.BlockSpec((2, bn_rows, d), lambda i: (0, i, 0)),
             _rows(bn_rows, d),
             _bcast((d, d)), _bcast((1, d))],
            [_rows(bn_rows, d), pl.BlockSpec((1, 2, d), lambda i: (i, 0, 0))],
            [jax.ShapeDtypeStruct((n, d), F32),
             jax.ShapeDtypeStruct((gn, 2, d), F32)],
            s_acc, den_p, h, Wout[l], bout[l].reshape(1, d))
        mu2 = jnp.sum(st2[:, 0, :], axis=0) / n
        var2 = jnp.sum(st2[:, 1, :], axis=0) / n - mu2 * mu2

        mu_st = jnp.stack([mu1, mu2], axis=0)
        var_st = jnp.stack([var1, var2], axis=0)
        dhalf = Wsg1.shape[2]
        a1 = jnp.concatenate([Wsg1[l, :d], Wsg1[l, d:2 * d]], axis=1)
        eb = jnp.stack(
            [jnp.pad(Wsg1[l, 2 * d], (0, d - dhalf)),
             jnp.pad(bsg1[l], (0, d - dhalf))], axis=0)

        h, h_mic = _call(
            _k_node3, gn,
            [_rows(bn_rows, d), _rows(bn_rows, d), _bcast((2, d)),
             _bcast((2, d)), _rows(bn_rows, d), _bcast((1, 1)),
             _bcast((d, 2 * dhalf)), _bcast((2, d)),
             _bcast((dhalf, 1)), _bcast((1, 1))],
            [_rows(bn_rows, d), _rows(bn_rows, d)],
            [jax.ShapeDtypeStruct((n, d), F32),
             jax.ShapeDtypeStruct((n, d), F32)],
            mac, z, mu_st, var_st, h,
            graph_entropy.reshape(1, 1), a1, eb, Wsg2[l],
            bsg2[l].reshape(1, 1))

        hm_src, hm_dst = _sc_dual_gather(h_mic, src, dst)
        e = _call(
            _k_edge_mlp, ge,
            [_rows(be_rows, d), _rows(be_rows, d), _rows(be_rows, d),
             _bcast((3 * d, d)), _bcast((1, d)), _bcast((d, d)),
             _bcast((1, d)), _bcast((3 * d, d)), _bcast((1, d))],
            [_rows(be_rows, d)],
            [jax.ShapeDtypeStruct((e_cnt, d), F32)],
            hm_src, hm_dst, e, Wm1[l], bm1[l].reshape(1, d), Wm2[l],
            bm2[l].reshape(1, d), Wg[l], bg[l].reshape(1, d))
    return h


# speedup vs baseline: 1.0433x; 1.0433x over previous
import functools

import jax
import jax.numpy as jnp
from jax import lax
from jax.experimental import pallas as pl
from jax.experimental.pallas import tpu as pltpu
from jax.experimental.pallas import tpu_sc as plsc

F32 = jnp.float32
I32 = jnp.int32

_NC = 2
_NS = 16
_NW = _NC * _NS


def _sc_mesh():
    return plsc.VectorSubcoreMesh(core_axis_name="c", subcore_axis_name="s",
                                  num_cores=_NC, num_subcores=_NS)


def _chunk_of(per_worker, cap=448):
    for c in range(min(per_worker, cap), 0, -1):
        if per_worker % c == 0 and c % 8 == 0:
            return c
    return per_worker


def _pad_rows(n):
    per = -(-n // (_NS * 8)) * 8
    return per * _NS


def _pick_block(n, target):
    b = min(n, target)
    while n % b or b % 8:
        b -= 1
    return b


def _ln(v):
    m = jnp.mean(v, axis=-1, keepdims=True)
    var = jnp.mean((v - m) ** 2, axis=-1, keepdims=True)
    return (v - m) * lax.rsqrt(var + 1e-5)


def _dot(a, b):
    return jnp.dot(a, b, preferred_element_type=F32)


def _k_node_enc(x_ref, wn_ref, bn_ref, deg_ref, h_ref, ent_ref):
    h = _ln(_dot(x_ref[...], wn_ref[...]) + bn_ref[...])
    h_ref[...] = h
    deg = deg_ref[:, 0]
    p = 1.0 / (deg + 1e-6)
    s = jnp.sum(deg * (-p * jnp.log(p + 1e-6)))
    ent_ref[0, 0, :] = jnp.broadcast_to(s, (ent_ref.shape[-1],))


def _k_edge_enc(ea_ref, we_ref, be_ref, e_ref):
    e_ref[...] = _ln(_dot(ea_ref[...], we_ref[...]) + be_ref[...])


def _k_node1(h_ref, agg_ref, deg_ref, ws_ref, wnn_ref, sb_ref,
             mac_ref, st_ref):
    h = h_ref[...]
    agg = (agg_ref[0] + agg_ref[1]) / (deg_ref[...] + 1e-6)
    mac = _dot(h, ws_ref[...]) + _dot(agg, wnn_ref[...]) + sb_ref[...]
    mac_ref[...] = mac
    st_ref[0, 0, :] = jnp.sum(mac, axis=0)
    st_ref[0, 1, :] = jnp.sum(mac * mac, axis=0)


def _k_edge_score(e_ref, hs_ref, hd_ref, we_ref, wq_ref, wk_ref, wv_ref,
                  g_ref, gt_ref, t_ref, sc_ref, mx_ref):
    ee = _dot(e_ref[...], we_ref[...])
    hs = hs_ref[...]
    q = _dot(hd_ref[...], wq_ref[...])
    k = _dot(hs, wk_ref[...]) + ee
    v = _dot(hs, wv_ref[...]) + ee
    sc = _dot(q * k, g_ref[...]) * 0.25
    t_ref[...] = v
    sc_ref[...] = sc
    mx8 = jnp.max(sc, axis=0, keepdims=True)
    mx_ref[0] = _dot(mx8, gt_ref[...])


def _k_edge_exp(t_ref, sc_ref, gmx_ref, gt_ref, msg_ref, z_ref):
    z8 = jnp.exp(sc_ref[...] - gmx_ref[...])
    z = _dot(z8, gt_ref[...])
    msg_ref[...] = z * t_ref[...]
    z_ref[...] = z


def _k_node2(s_ref, den_ref, h_ref, wo_ref, bo_ref, z_ref, st_ref):
    s = s_ref[0] + s_ref[1]
    den = den_ref[0] + den_ref[1] + 1e-16
    outn = _dot(s / den, wo_ref[...]) + bo_ref[...]
    z = outn + h_ref[...]
    z_ref[...] = z
    st_ref[0, 0, :] = jnp.sum(z, axis=0)
    st_ref[0, 1, :] = jnp.sum(z * z, axis=0)


def _k_node3(mac_ref, z_ref, mu_ref, var_ref, h_ref, ent_ref,
             a1_ref, eb_ref, w2_ref, b2_ref,
             hnew_ref, hmic_ref):
    mu = mu_ref[...]
    var = var_ref[...]
    hmac = jax.nn.gelu((mac_ref[...] - mu[0:1, :]) *
                       lax.rsqrt(var[0:1, :] + 1e-5)) + h_ref[...]
    hmic = jax.nn.gelu((z_ref[...] - mu[1:2, :]) *
                       lax.rsqrt(var[1:2, :] + 1e-5))
    dh2 = a1_ref.shape[1] // 2
    gi = (_dot(hmac, a1_ref[:, :dh2]) + _dot(hmic, a1_ref[:, dh2:])
          + ent_ref[0, 0] * eb_ref[0:1, :dh2] + eb_ref[1:2, :dh2])
    sg = jax.nn.silu(gi)
    a = jax.nn.sigmoid(_dot(sg, w2_ref[...]) + b2_ref[...])
    hnew_ref[...] = a * hmac + (1.0 - a) * hmic
    hmic_ref[...] = hmic


def _k_edge_mlp(hs_ref, hd_ref, e_ref, w1_ref, b1_ref, w2_ref, b2_ref,
                wg_ref, bg_ref, eo_ref):
    hs = hs_ref[...]
    hd = hd_ref[...]
    e = e_ref[...]
    d = e.shape[1]
    w1 = w1_ref[...]
    wg = wg_ref[...]
    c1 = (_dot(hs, w1[:d]) + _dot(hd, w1[d:2 * d]) + _dot(e, w1[2 * d:])
          + b1_ref[...])
    u = jax.nn.gelu(_ln(c1))
    upd = _dot(u, w2_ref[...]) + b2_ref[...]
    g = jax.nn.sigmoid(_dot(hs, wg[:d]) + _dot(hd, wg[d:2 * d])
                       + _dot(e, wg[2 * d:]) + bg_ref[...])
    eo_ref[...] = _ln(upd * g + e)


def _rows(block, ncols):
    return pl.BlockSpec((block, ncols), lambda i: (i, 0))


def _bcast(shape):
    nd = len(shape)
    return pl.BlockSpec(shape, lambda i: (0,) * nd)


def _call(body, grid, in_specs, out_specs, out_shapes, *args):
    r = pl.pallas_call(
        body,
        grid=(grid,),
        in_specs=in_specs,
        out_specs=out_specs,
        out_shape=out_shapes,
    )(*args)
    return r[0] if len(out_shapes) == 1 else r


def _sc_gather_scatter(h, src, dst, zeros):
    n, d = h.shape
    e = src.shape[0]
    ew = e // _NW
    c = _chunk_of(ew, cap=96)
    iters = ew // c
    n_pad = zeros.shape[0]
    rps = n_pad // _NS

    def body(h_hbm, src_hbm, dst_hbm, zer_hbm, hs_out, hd_out, agg_out,
             sidx0, didx0, sidx1, didx1, r00, r01, r10, r11, acc,
             g00, g01, g10, g11, i00, i01, i10, i11):
        ci = lax.axis_index("c")
        si = lax.axis_index("s")
        wid = si * _NC + ci
        pltpu.sync_copy(zer_hbm.at[pl.ds(si * rps, rps)],
                        acc.at[pl.ds(si * rps, rps)])
        plsc.subcore_barrier()
        base0 = wid * ew

        def one(b, sidx, didx, rs, rd, gs, gd, js, jd):
            ais = pltpu.async_copy(src_hbm.at[pl.ds(b, c)], sidx, js)
            aid = pltpu.async_copy(dst_hbm.at[pl.ds(b, c)], didx, jd)
            ais.wait()
            cs = pltpu.async_copy(h_hbm.at[sidx], rs, gs)
            aid.wait()
            cd = pltpu.async_copy(h_hbm.at[didx], rd, gd)
            return cs, cd

        def fin(b, didx, rs, rd, cs, cd):
            cs.wait()
            pltpu.sync_copy(rs, hs_out.at[pl.ds(b, c)])
            pltpu.sync_copy(rs, acc.at[didx], add=True)
            cd.wait()
            pltpu.sync_copy(rd, hd_out.at[pl.ds(b, c)])

        def step(j, carry):
            b0 = base0 + (2 * j) * c
            b1 = b0 + c
            cs0, cd0 = one(b0, sidx0, didx0, r00, r01, g00, g01, i00, i01)
            cs1, cd1 = one(b1, sidx1, didx1, r10, r11, g10, g11, i10, i11)
            fin(b0, didx0, r00, r01, cs0, cd0)
            fin(b1, didx1, r10, r11, cs1, cd1)
            return carry

        lax.fori_loop(0, iters // 2, step, 0)
        if iters % 2:
            bt = base0 + (iters - 1) * c
            cs0, cd0 = one(bt, sidx0, didx0, r00, r01, g00, g01, i00, i01)
            fin(bt, didx0, r00, r01, cs0, cd0)
        plsc.subcore_barrier()
        pltpu.sync_copy(acc.at[pl.ds(si * rps, rps)],
                        agg_out.at[ci].at[pl.ds(si * rps, rps)])

    f = pl.kernel(
        body,
        out_type=[jax.ShapeDtypeStruct((e, d), F32),
                  jax.ShapeDtypeStruct((e, d), F32),
                  jax.ShapeDtypeStruct((_NC, n_pad, d), F32)],
        mesh=_sc_mesh(),
        scratch_types=[pltpu.VMEM((c,), I32), pltpu.VMEM((c,), I32),
                       pltpu.VMEM((c,), I32), pltpu.VMEM((c,), I32),
                       pltpu.VMEM((c, d), F32), pltpu.VMEM((c, d), F32),
                       pltpu.VMEM((c, d), F32), pltpu.VMEM((c, d), F32),
                       pltpu.VMEM_SHARED((n_pad, d), F32),
                       pltpu.SemaphoreType.DMA, pltpu.SemaphoreType.DMA,
                       pltpu.SemaphoreType.DMA, pltpu.SemaphoreType.DMA,
                       pltpu.SemaphoreType.DMA, pltpu.SemaphoreType.DMA,
                       pltpu.SemaphoreType.DMA, pltpu.SemaphoreType.DMA],
    )
    return f(h, src, dst, zeros)


def _sc_scatter(p, dst, zeros):
    n_pad, w = zeros.shape
    e, _ = p.shape
    ew = e // _NW
    c = _chunk_of(ew, cap=200)
    iters = ew // c
    rps = n_pad // _NS

    def body(p_hbm, dst_hbm, zer_hbm, out, pbuf0, didx0, didx1,
             acc, s0, si0, si1):
        ci = lax.axis_index("c")
        si = lax.axis_index("s")
        wid = si * _NC + ci
        pltpu.sync_copy(zer_hbm.at[pl.ds(si * rps, rps)],
                        acc.at[pl.ds(si * rps, rps)])
        plsc.subcore_barrier()
        base0 = wid * ew

        def step(j, carry):
            b0 = base0 + (2 * j) * c
            b1 = b0 + c
            ai0 = pltpu.async_copy(dst_hbm.at[pl.ds(b0, c)], didx0, si0)
            ai1 = pltpu.async_copy(dst_hbm.at[pl.ds(b1, c)], didx1, si1)
            ap0 = pltpu.async_copy(p_hbm.at[pl.ds(b0, c)], pbuf0, s0)
            ai0.wait()
            ap0.wait()
            pltpu.sync_copy(pbuf0, acc.at[didx0], add=True)
            ap1 = pltpu.async_copy(p_hbm.at[pl.ds(b1, c)], pbuf0, s0)
            ai1.wait()
            ap1.wait()
            pltpu.sync_copy(pbuf0, acc.at[didx1], add=True)
            return carry

        lax.fori_loop(0, iters // 2, step, 0)
        plsc.subcore_barrier()
        pltpu.sync_copy(acc.at[pl.ds(si * rps, rps)],
                        out.at[ci].at[pl.ds(si * rps, rps)])

    f = pl.kernel(
        body,
        out_type=[jax.ShapeDtypeStruct((_NC, n_pad, w), F32)],
        mesh=_sc_mesh(),
        scratch_types=[pltpu.VMEM((c, w), F32),
                       pltpu.VMEM((c,), I32), pltpu.VMEM((c,), I32),
                       pltpu.VMEM_SHARED((n_pad, w), F32),
                       pltpu.SemaphoreType.DMA,
                       pltpu.SemaphoreType.DMA, pltpu.SemaphoreType.DMA],
    )
    return f(p, dst, zeros)[0]


def _sc_dual_gather(tab, src, dst):
    n, d = tab.shape
    e = src.shape[0]
    ew = e // _NW
    c = _chunk_of(ew, cap=200)
    iters = ew // c

    def body(t_hbm, src_hbm, dst_hbm, s_out, d_out,
             sidx0, didx0, sidx1, didx1, r00, r01, r10, r11,
             g00, g01, g10, g11, i00, i01, i10, i11):
        ci = lax.axis_index("c")
        si = lax.axis_index("s")
        wid = si * _NC + ci
        base0 = wid * ew

        def step(j, carry):
            b0 = base0 + (2 * j) * c
            b1 = b0 + c
            a00 = pltpu.async_copy(src_hbm.at[pl.ds(b0, c)], sidx0, i00)
            a01 = pltpu.async_copy(dst_hbm.at[pl.ds(b0, c)], didx0, i01)
            a10 = pltpu.async_copy(src_hbm.at[pl.ds(b1, c)], sidx1, i10)
            a11 = pltpu.async_copy(dst_hbm.at[pl.ds(b1, c)], didx1, i11)
            a00.wait()
            c00 = pltpu.async_copy(t_hbm.at[sidx0], r00, g00)
            a01.wait()
            c01 = pltpu.async_copy(t_hbm.at[didx0], r01, g01)
            a10.wait()
            c10 = pltpu.async_copy(t_hbm.at[sidx1], r10, g10)
            a11.wait()
            c11 = pltpu.async_copy(t_hbm.at[didx1], r11, g11)
            c00.wait()
            pltpu.sync_copy(r00, s_out.at[pl.ds(b0, c)])
            c01.wait()
            pltpu.sync_copy(r01, d_out.at[pl.ds(b0, c)])
            c10.wait()
            pltpu.sync_copy(r10, s_out.at[pl.ds(b1, c)])
            c11.wait()
            pltpu.sync_copy(r11, d_out.at[pl.ds(b1, c)])
            return carry

        lax.fori_loop(0, iters // 2, step, 0)

    f = pl.kernel(
        body,
        out_type=[jax.ShapeDtypeStruct((e, d), F32),
                  jax.ShapeDtypeStruct((e, d), F32)],
        mesh=_sc_mesh(),
        scratch_types=[pltpu.VMEM((c,), I32), pltpu.VMEM((c,), I32),
                       pltpu.VMEM((c,), I32), pltpu.VMEM((c,), I32),
                       pltpu.VMEM((c, d), F32), pltpu.VMEM((c, d), F32),
                       pltpu.VMEM((c, d), F32), pltpu.VMEM((c, d), F32),
                       pltpu.SemaphoreType.DMA, pltpu.SemaphoreType.DMA,
                       pltpu.SemaphoreType.DMA, pltpu.SemaphoreType.DMA,
                       pltpu.SemaphoreType.DMA, pltpu.SemaphoreType.DMA,
                       pltpu.SemaphoreType.DMA, pltpu.SemaphoreType.DMA],
    )
    return f(tab, src, dst)


def kernel(x, edge_index, edge_attr, Wn, bn, We_enc, be_enc, sage_Ws,
           sage_Wn, sage_b, WQ, WK, WV, WE, Wout, bout, Wm1, bm1, Wm2,
           bm2, Wg, bg, Wsg1, bsg1, Wsg2, bsg2):
    n, d = x.shape
    e_cnt = edge_index.shape[1]
    nheads = 8
    dh = d // nheads
    src = edge_index[0]
    dst = edge_index[1]

    bn_rows = _pick_block(n, 1024)
    be_rows = _pick_block(e_cnt, 2048)
    gn = n // bn_rows
    ge = e_cnt // be_rows

    hid = jnp.arange(d, dtype=jnp.int32) // dh
    g_sum = (hid[:, None] == jnp.arange(nheads)[None, :]).astype(F32)
    g_bc = g_sum.T

    n_pad = _pad_rows(n)
    z128 = jnp.zeros((n_pad, d), F32)
    ones_e = jnp.ones((e_cnt,), F32)
    deg_src = jax.ops.segment_sum(ones_e, src, n).reshape(n, 1)
    deg_dst = jax.ops.segment_sum(ones_e, dst, n).reshape(n, 1)

    h, ent_part = _call(
        _k_node_enc, gn,
        [_rows(bn_rows, d), _bcast((d, d)), _bcast((1, d)),
         _rows(bn_rows, 1)],
        [_rows(bn_rows, d), pl.BlockSpec((1, 1, d), lambda i: (i, 0, 0))],
        [jax.ShapeDtypeStruct((n, d), F32),
         jax.ShapeDtypeStruct((gn, 1, d), F32)],
        x, Wn, bn.reshape(1, d), deg_src)
    graph_entropy = jnp.sum(ent_part[:, 0, 0]) / e_cnt

    ed = edge_attr.shape[1]
    e = _call(
        _k_edge_enc, ge,
        [_rows(be_rows, ed), _bcast((ed, d)), _bcast((1, d))],
        [_rows(be_rows, d)],
        [jax.ShapeDtypeStruct((e_cnt, d), F32)],
        edge_attr, We_enc, be_enc.reshape(1, d))

    nlayers = sage_Ws.shape[0]
    for l in range(nlayers):
        h_src, h_dst, agg_p = _sc_gather_scatter(h, src, dst, z128)
        mac, st = _call(
            _k_node1, gn,
            [_rows(bn_rows, d),
             pl.BlockSpec((2, bn_rows, d), lambda i: (0, i, 0)),
             _rows(bn_rows, 1),
             _bcast((d, d)), _bcast((d, d)), _bcast((1, d))],
            [_rows(bn_rows, d),
             pl.BlockSpec((1, 2, d), lambda i: (i, 0, 0))],
            [jax.ShapeDtypeStruct((n, d), F32),
             jax.ShapeDtypeStruct((gn, 2, d), F32)],
            h, agg_p, deg_dst, sage_Ws[l], sage_Wn[l],
            sage_b[l].reshape(1, d))
        mu1 = jnp.sum(st[:, 0, :], axis=0) / n
        var1 = jnp.sum(st[:, 1, :], axis=0) / n - mu1 * mu1

        t, sc, mxp = _call(
            _k_edge_score, ge,
            [_rows(be_rows, d), _rows(be_rows, d), _rows(be_rows, d),
             _bcast((d, d)), _bcast((d, d)), _bcast((d, d)),
             _bcast((d, d)),
             _bcast((d, nheads)), _bcast((nheads, d))],
            [_rows(be_rows, d), _rows(be_rows, nheads),
             pl.BlockSpec((1, 1, d), lambda i: (i, 0, 0))],
            [jax.ShapeDtypeStruct((e_cnt, d), F32),
             jax.ShapeDtypeStruct((e_cnt, nheads), F32),
             jax.ShapeDtypeStruct((ge, 1, d), F32)],
            e, h_src, h_dst, WE[l], WQ[l], WK[l], WV[l], g_sum, g_bc)
        gmx8 = jnp.max(mxp[:, 0, ::dh], axis=0).reshape(1, nheads)
        msg, zx = _call(
            _k_edge_exp, ge,
            [_rows(be_rows, d), _rows(be_rows, nheads),
             _bcast((1, nheads)), _bcast((nheads, d))],
            [_rows(be_rows, d), _rows(be_rows, d)],
            [jax.ShapeDtypeStruct((e_cnt, d), F32),
             jax.ShapeDtypeStruct((e_cnt, d), F32)],
            t, sc, gmx8, g_bc)

        s_acc = _sc_scatter(msg, dst, z128)
        den_p = _sc_scatter(zx, dst, z128)

        z, st2 = _call(
            _k_node2, gn,
            [pl.BlockSpec((2, bn_rows, d), lambda i: (0, i, 0)),
             pl.BlockSpec((2, bn_rows, d), lambda i: (0, i, 0)),
             _rows(bn_rows, d),
             _bcast((d, d)), _bcast((1, d))],
            [_rows(bn_rows, d), pl.BlockSpec((1, 2, d), lambda i: (i, 0, 0))],
            [jax.ShapeDtypeStruct((n, d), F32),
             jax.ShapeDtypeStruct((gn, 2, d), F32)],
            s_acc, den_p, h, Wout[l], bout[l].reshape(1, d))
        mu2 = jnp.sum(st2[:, 0, :], axis=0) / n
        var2 = jnp.sum(st2[:, 1, :], axis=0) / n - mu2 * mu2

        mu_st = jnp.stack([mu1, mu2], axis=0)
        var_st = jnp.stack([var1, var2], axis=0)
        dhalf = Wsg1.shape[2]
        a1 = jnp.concatenate([Wsg1[l, :d], Wsg1[l, d:2 * d]], axis=1)
        eb = jnp.stack(
            [jnp.pad(Wsg1[l, 2 * d], (0, d - dhalf)),
             jnp.pad(bsg1[l], (0, d - dhalf))], axis=0)

        h, h_mic = _call(
            _k_node3, gn,
            [_rows(bn_rows, d), _rows(bn_rows, d), _bcast((2, d)),
             _bcast((2, d)), _rows(bn_rows, d), _bcast((1, 1)),
             _bcast((d, 2 * dhalf)), _bcast((2, d)),
             _bcast((dhalf, 1)), _bcast((1, 1))],
            [_rows(bn_rows, d), _rows(bn_rows, d)],
            [jax.ShapeDtypeStruct((n, d), F32),
             jax.ShapeDtypeStruct((n, d), F32)],
            mac, z, mu_st, var_st, h,
            graph_entropy.reshape(1, 1), a1, eb, Wsg2[l],
            bsg2[l].reshape(1, 1))

        hm_src, hm_dst = _sc_dual_gather(h_mic, src, dst)
        e = _call(
            _k_edge_mlp, ge,
            [_rows(be_rows, d), _rows(be_rows, d), _rows(be_rows, d),
             _bcast((3 * d, d)), _bcast((1, d)), _bcast((d, d)),
             _bcast((1, d)), _bcast((3 * d, d)), _bcast((1, d))],
            [_rows(be_rows, d)],
            [jax.ShapeDtypeStruct((e_cnt, d), F32)],
            hm_src, hm_dst, e, Wm1[l], bm1[l].reshape(1, d), Wm2[l],
            bm2[l].reshape(1, d), Wg[l], bg[l].reshape(1, d))
    return h
